# Initial kernel scaffold; baseline (speedup 1.0000x reference)
import jax
import jax.numpy as jnp
from jax.experimental import pallas as pl


def kernel(features, edge_index, edge_features, W1, b1, W2, b2, W3, b3, W4, b4):
    raise NotImplementedError("write your pallas kernel here")


# speedup vs baseline: 1.5025x; 1.5025x over previous
import functools

import jax
import jax.numpy as jnp
from jax import lax
from jax.experimental import pallas as pl
from jax.experimental.pallas import tpu as pltpu
from jax.experimental.pallas import tpu_sc as plsc

N = 10000
E = 320000
D = 128
DE = 16

_NC, _NS = 2, 16
K = 128
NCH = 2512
EPAD = NCH * K
_CPT = NCH // _NS
HALF = 5120
NPH = 5248
_RPT = NPH // _NS

_PREC = lax.Precision.HIGHEST


def _proj_body(f_ref, wsT_ref, wdT_ref, ps_ref, pd_ref):
    f = f_ref[...]
    ps_ref[...] = lax.dot(f, wsT_ref[...], precision=_PREC,
                          preferred_element_type=jnp.float32)
    pd_ref[...] = lax.dot(f, wdT_ref[...], precision=_PREC,
                          preferred_element_type=jnp.float32)


def _edge_body(e_ref, weT_ref, b1_ref, pe_ref):
    pe_ref[...] = lax.dot(e_ref[...], weT_ref[...],
                          preferred_element_type=jnp.float32) + b1_ref[...]


def _out_body(f_ref, a_ref, w2T_ref, w3aT_ref,
              w3bT_ref, w4T_ref, b3_ref, b4_ref, z_ref):
    f = f_ref[...]
    s = f + lax.dot(a_ref[...], w2T_ref[...], precision=_PREC,
                    preferred_element_type=jnp.float32)
    t = lax.dot(f, w3aT_ref[...], precision=_PREC,
                preferred_element_type=jnp.float32)
    t = t + lax.dot(s, w3bT_ref[...], precision=_PREC,
                    preferred_element_type=jnp.float32) + b3_ref[...]
    t = jnp.maximum(t, 0.0)
    z_ref[...] = lax.dot(t, w4T_ref[...], precision=_PREC,
                         preferred_element_type=jnp.float32) + b4_ref[...]


_BN = 2000
_BE = 8000


def _proj_call(f, wsT, wdT):
    w_spec = pl.BlockSpec((D, D), lambda i: (0, 0))
    return pl.pallas_call(
        _proj_body,
        grid=(N // _BN,),
        in_specs=[pl.BlockSpec((_BN, D), lambda i: (i, 0)), w_spec, w_spec],
        out_specs=[pl.BlockSpec((_BN, D), lambda i: (i, 0))] * 2,
        out_shape=[jax.ShapeDtypeStruct((N, D), jnp.float32)] * 2,
    )(f, wsT, wdT)


def _edge_call(e, weT, b1r):
    return pl.pallas_call(
        _edge_body,
        grid=(E // _BE,),
        in_specs=[pl.BlockSpec((_BE, DE), lambda i: (i, 0)),
                  pl.BlockSpec((DE, D), lambda i: (0, 0)),
                  pl.BlockSpec((1, D), lambda i: (0, 0))],
        out_specs=pl.BlockSpec((_BE, D), lambda i: (i, 0)),
        out_shape=jax.ShapeDtypeStruct((E, D), jnp.float32),
    )(e, weT, b1r)


def _out_call(f, a, w2T, w3aT, w3bT, w4T, b3r, b4r):
    blk = pl.BlockSpec((_BN, D), lambda i: (i, 0))
    w_spec = pl.BlockSpec((D, D), lambda i: (0, 0))
    b_spec = pl.BlockSpec((1, D), lambda i: (0, 0))
    return pl.pallas_call(
        _out_body,
        grid=(N // _BN,),
        in_specs=[blk, blk,
                  w_spec, w_spec, w_spec, w_spec, b_spec, b_spec],
        out_specs=blk,
        out_shape=jax.ShapeDtypeStruct((N, D), jnp.float32),
    )(f, a, w2T, w3aT, w3bT, w4T, b3r, b4r)


def _sc_body(ps_hbm, pd_hbm, pe_hbm, src_hbm, dst_hbm, loc_hbm, x_hbm,
             out_hbm,
             bufa, bufb, bufc, idx_s, idx_d, idx_l, xv,
             acc, sem_a, sem_b):
    c = lax.axis_index("c")
    s = lax.axis_index("s")

    pltpu.sync_copy(x_hbm, xv)

    zero16 = jnp.zeros((16,), jnp.float32)

    def _fill(r, _):
        for j in range(D // 16):
            bufa[r, pl.ds(j * 16, 16)] = zero16
        return 0

    lax.fori_loop(0, K, _fill, 0)

    row0 = s * _RPT
    off = 0
    for sz in (K, K, _RPT - 2 * K):
        pltpu.sync_copy(bufa.at[pl.ds(0, sz)],
                        acc.at[pl.ds(row0 + off, sz)])
        off += sz
    plsc.subcore_barrier()

    def _chunk(t, _):
        chunk = s + _NS * t
        base = chunk * K
        pltpu.sync_copy(src_hbm.at[pl.ds(base, K)], idx_s)
        pltpu.sync_copy(dst_hbm.at[pl.ds(base, K)], idx_d)
        pltpu.sync_copy(loc_hbm.at[pl.ds(c * EPAD + base, K)], idx_l)
        cp_a = pltpu.async_copy(ps_hbm.at[idx_s], bufa, sem_a)
        cp_b = pltpu.async_copy(pd_hbm.at[idx_d], bufb, sem_b)
        pltpu.sync_copy(pe_hbm.at[pl.ds(base, K)], bufc)
        cp_a.wait()
        cp_b.wait()

        def _relu_row(r, _):
            for j in range(D // 16):
                sl = pl.ds(j * 16, 16)
                v = bufa[r, sl] + bufb[r, sl] + bufc[r, sl]
                bufc[r, sl] = jnp.maximum(v, 0.0) + xv[sl]
            return 0

        lax.fori_loop(0, K, _relu_row, 0)
        pltpu.sync_copy(bufc, acc.at[idx_l], add=True)
        return 0

    lax.fori_loop(0, _CPT, _chunk, 0)
    plsc.subcore_barrier()

    out_row = c * NPH + row0
    pltpu.sync_copy(acc.at[pl.ds(row0, _RPT)],
                    out_hbm.at[pl.ds(out_row, _RPT)])


@functools.cache
def _make_sc_call():
    return pl.kernel(
        _sc_body,
        out_type=jax.ShapeDtypeStruct((_NC * NPH, D), jnp.float32),
        mesh=plsc.VectorSubcoreMesh(core_axis_name="c", subcore_axis_name="s",
                                    num_cores=_NC, num_subcores=_NS),
        scratch_types=[
            pltpu.VMEM((K, D), jnp.float32),
            pltpu.VMEM((K, D), jnp.float32),
            pltpu.VMEM((K, D), jnp.float32),
            pltpu.VMEM((K,), jnp.int32),
            pltpu.VMEM((K,), jnp.int32),
            pltpu.VMEM((K,), jnp.int32),
            pltpu.VMEM((D,), jnp.float32),
            pltpu.VMEM_SHARED((NPH, D), jnp.float32),
            pltpu.SemaphoreType.DMA,
            pltpu.SemaphoreType.DMA,
        ],
    )


def kernel(features, edge_index, edge_features, W1, b1, W2, b2, W3, b3, W4, b4):
    src = edge_index[0].astype(jnp.int32)
    dst = edge_index[1].astype(jnp.int32)
    w1sT = W1[:, :D].T
    w1eT = W1[:, D:D + DE].T
    w1dT = W1[:, D + DE:].T

    ps, pd = _proj_call(features, w1sT, w1dT)
    pe = _edge_call(edge_features, w1eT, b1.reshape(1, D))

    x = jnp.linalg.solve(W2, b2)
    for _ in range(2):
        x = x + jnp.linalg.solve(W2, b2 - W2 @ x)

    pad = EPAD - E
    src_p = jnp.concatenate([src, jnp.zeros((pad,), jnp.int32)])
    dst_p = jnp.concatenate([dst, jnp.zeros((pad,), jnp.int32)])
    pe_p = jnp.concatenate([pe, jnp.zeros((pad, D), jnp.float32)])
    loc0 = jnp.where(src < HALF, src, HALF)
    loc1 = jnp.where(src >= HALF, src - HALF, HALF)
    trash = jnp.full((pad,), HALF, jnp.int32)
    locs = jnp.concatenate([loc0, trash, loc1, trash])
    part = _make_sc_call()(ps, pd, pe_p, src_p, dst_p, locs, x)
    a_full = jnp.concatenate([part[:HALF], part[NPH:NPH + (N - HALF)]])

    return _out_call(features, a_full,
                     W2.T, W3[:, :D].T, W3[:, D:].T, W4.T,
                     b3.reshape(1, D), b4.reshape(1, D))
